# baseline (speedup 1.0000x reference)
import functools

import jax
import jax.numpy as jnp
from jax import lax
from jax.experimental import pallas as pl
from jax.experimental.pallas import tpu as pltpu
from jax.experimental.pallas import tpu_sc as plsc

N = 10240
E = 163840
D = 128
OD2 = 256
NTOW = 2
NC, NS = 2, 16
W = 64
EPT = E // NS
NWIN = EPT // W
RPT = N // NS
DEPTH = 4
CH = 16
NCHUNK = NWIN // CH
BNS = 0.9999950000374996


def _deg_body(dsts2, ones_h, zrow_h, deg, dstt, onesv, zv, acc):
    c = lax.axis_index("c")
    s = lax.axis_index("s")
    pltpu.sync_copy(ones_h, onesv)
    pltpu.sync_copy(zrow_h, zv)
    pltpu.sync_copy(zv, acc.at[pl.ds(s * RPT, RPT)])
    tb = (c * NS + s) * NWIN
    pltpu.sync_copy(dsts2.at[pl.ds(tb, NWIN)], dstt)
    plsc.subcore_barrier()

    def body(w, carry):
        pltpu.sync_copy(onesv, acc.at[dstt.at[w]], add=True)
        return carry

    lax.fori_loop(0, NWIN, body, 0)
    plsc.subcore_barrier()
    pltpu.sync_copy(acc.at[pl.ds(s * RPT, RPT)],
                    deg.at[pl.ds(c * N + s * RPT, RPT)])


def _spmm_body(xs, srcs2, dsts2, y, srct, dstt, rows, acc, sems):
    c = lax.axis_index("c")
    s = lax.axis_index("s")
    rbase = s * RPT
    pltpu.sync_copy(xs.at[pl.ds(c * N + rbase, RPT)],
                    acc.at[pl.ds(rbase, RPT)])
    plsc.subcore_barrier()
    tb = (c * NS + s) * NWIN

    def chunk_body(ch, carry):
        pltpu.sync_copy(srcs2.at[pl.ds(tb + ch * CH, CH)], srct)
        pltpu.sync_copy(dsts2.at[pl.ds(tb + ch * CH, CH)], dstt)
        for k in range(DEPTH - 1):
            pltpu.async_copy(xs.at[srct.at[k]], rows[k], sems[k])
        for w in range(CH):
            b = w % DEPTH
            nb = (b + DEPTH - 1) % DEPTH
            if w + DEPTH - 1 < CH:
                pltpu.async_copy(xs.at[srct.at[w + DEPTH - 1]],
                                 rows[nb], sems[nb])
            pltpu.make_async_copy(xs.at[srct.at[w]], rows[b], sems[b]).wait()
            pltpu.sync_copy(rows[b], acc.at[dstt.at[w]], add=True)
        return carry

    lax.fori_loop(0, NCHUNK, chunk_body, 0)
    plsc.subcore_barrier()
    pltpu.sync_copy(acc.at[pl.ds(rbase, RPT)], y.at[pl.ds(c * N + rbase, RPT)])


@functools.cache
def _sc_kernels():
    mesh = plsc.VectorSubcoreMesh(
        core_axis_name="c", subcore_axis_name="s",
        num_cores=NC, num_subcores=NS)
    deg = pl.kernel(
        _deg_body,
        out_type=jax.ShapeDtypeStruct((NTOW * N,), jnp.float32),
        mesh=mesh,
        scratch_types=[
            pltpu.VMEM((NWIN, W), jnp.int32),
            pltpu.VMEM((W,), jnp.float32),
            pltpu.VMEM((RPT,), jnp.float32),
            pltpu.VMEM_SHARED((N,), jnp.float32),
        ],
    )
    spmm = pl.kernel(
        _spmm_body,
        out_type=jax.ShapeDtypeStruct((NTOW * N, D), jnp.float32),
        mesh=mesh,
        scratch_types=[
            pltpu.VMEM((CH, W), jnp.int32),
            pltpu.VMEM((CH, W), jnp.int32),
            [pltpu.VMEM((W, D), jnp.float32) for _ in range(DEPTH)],
            pltpu.VMEM_SHARED((N, D), jnp.float32),
            [pltpu.SemaphoreType.DMA for _ in range(DEPTH)],
        ],
    )
    return deg, spmm


def _cell_body(cell, Wr1, br1, Wr2, br2, Wr3, br3, Wq, cv_out, q_out):
    x = cell[...]
    nrm = jnp.sqrt(jnp.sum(x * x, axis=1, keepdims=True))
    x = x / jnp.maximum(nrm, 1e-12)
    x = jnp.maximum(
        jnp.dot(x, Wr1[...], preferred_element_type=jnp.float32) + br1[...], 0.0)
    x = jnp.maximum(
        jnp.dot(x, Wr2[...], preferred_element_type=jnp.float32) + br2[...], 0.0)
    cv = jnp.dot(x, Wr3[...], preferred_element_type=jnp.float32) + br3[...]
    cv_out[...] = cv
    q_out[...] = jnp.dot(cv, Wq[...], preferred_element_type=jnp.float32)


def _pre_body(x, Wc1, deg, u1, dinv_out):
    dinv = 1.0 / jnp.sqrt(deg[...] + 1.0)
    t = jnp.dot(x[...], Wc1[...], preferred_element_type=jnp.float32)
    u1[...] = t * dinv
    dinv_out[...] = dinv


def _mid1_body(y, dinv, bc1, Wc2, u2):
    h = jnp.maximum(y[...] * dinv[...] + bc1[...], 0.0)
    u2[...] = jnp.dot(h, Wc2[...], preferred_element_type=jnp.float32) * dinv[...]


def _mid2_body(y, dinv, bc2, u3):
    h = jnp.maximum(y[...] * dinv[...] + bc2[...], 0.0)
    u3[...] = h * dinv[...]


def _head_body(y, dinv, Wc3, bc3, g2, bt2, Wk, Wv, q, ctx_out, pool_out):
    agg = y[...] * dinv[...]
    h3 = jnp.dot(agg, Wc3[...], preferred_element_type=jnp.float32) + bc3[...]
    h3 = jnp.maximum(g2[...] * h3 * BNS + bt2[...], 0.0)
    pool_out[...] = jnp.max(h3.reshape(32, 40, OD2), axis=1)
    K = jnp.dot(h3, Wk[...], preferred_element_type=jnp.float32)
    V = jnp.dot(h3, Wv[...], preferred_element_type=jnp.float32)
    qb = q[...]
    ctxs = []
    for h in range(4):
        Kh = K[:, h * 64:(h + 1) * 64].reshape(32, 40, 64)
        Vh = V[:, h * 64:(h + 1) * 64].reshape(32, 40, 64)
        qh = qb[:, h * 64:(h + 1) * 64]
        s = jnp.sum(Kh * qh[:, None, :], axis=-1) * 0.0625
        m = jnp.max(s, axis=-1, keepdims=True)
        e = jnp.exp(s - m)
        p = e / jnp.sum(e, axis=-1, keepdims=True)
        ctxs.append(jnp.sum(p[:, :, None] * Vh, axis=1))
    ctx_out[...] = jnp.concatenate(ctxs, axis=1)


def _final_body(ctx, pool, cv, Wg1p, Wg1, bg1, g4, bt4, Wf1, bf1, Wo, bo, out):
    ctxa = ctx[...]
    poola = pool[...]
    ctxs = ctxa[0:256] + ctxa[256:512]
    pools = poola[0:256] + poola[256:512]
    drug = (jnp.dot(ctxs, Wg1p[...], preferred_element_type=jnp.float32)
            + jnp.dot(pools, Wg1[...], preferred_element_type=jnp.float32)
            + 2.0 * bg1[...])
    xc = jnp.concatenate([drug, cv[...]], axis=1)
    nrm = jnp.sqrt(jnp.sum(xc * xc, axis=1, keepdims=True))
    xc = xc / jnp.maximum(nrm, 1e-12)
    xc = jnp.maximum(g4[...] * xc * BNS + bt4[...], 0.0)
    xc = jnp.maximum(
        jnp.dot(xc, Wf1[...], preferred_element_type=jnp.float32) + bf1[...], 0.0)
    out[...] = jnp.dot(xc, Wo[...], preferred_element_type=jnp.float32) + bo[...]


def _full(shape):
    return pl.BlockSpec(shape, lambda *_: tuple(0 for _ in shape))


_cell = pl.pallas_call(
    _cell_body,
    out_shape=[jax.ShapeDtypeStruct((256, OD2), jnp.float32),
               jax.ShapeDtypeStruct((256, OD2), jnp.float32)],
)

_pre = pl.pallas_call(
    _pre_body,
    grid=(NTOW * N // 256,),
    in_specs=[pl.BlockSpec((256, 78), lambda i: (i, 0)),
              _full((78, D)),
              pl.BlockSpec((256, 1), lambda i: (i, 0))],
    out_specs=[pl.BlockSpec((256, D), lambda i: (i, 0)),
               pl.BlockSpec((256, 1), lambda i: (i, 0))],
    out_shape=[jax.ShapeDtypeStruct((NTOW * N, D), jnp.float32),
               jax.ShapeDtypeStruct((NTOW * N, 1), jnp.float32)],
)

_mid1 = pl.pallas_call(
    _mid1_body,
    grid=(NTOW * N // 256,),
    in_specs=[pl.BlockSpec((256, D), lambda i: (i, 0)),
              pl.BlockSpec((256, 1), lambda i: (i, 0)),
              _full((1, D)),
              _full((D, D))],
    out_specs=pl.BlockSpec((256, D), lambda i: (i, 0)),
    out_shape=jax.ShapeDtypeStruct((NTOW * N, D), jnp.float32),
)

_mid2 = pl.pallas_call(
    _mid2_body,
    grid=(NTOW * N // 256,),
    in_specs=[pl.BlockSpec((256, D), lambda i: (i, 0)),
              pl.BlockSpec((256, 1), lambda i: (i, 0)),
              _full((1, D))],
    out_specs=pl.BlockSpec((256, D), lambda i: (i, 0)),
    out_shape=jax.ShapeDtypeStruct((NTOW * N, D), jnp.float32),
)

_head = pl.pallas_call(
    _head_body,
    grid=(16,),
    in_specs=[pl.BlockSpec((1280, D), lambda i: (i, 0)),
              pl.BlockSpec((1280, 1), lambda i: (i, 0)),
              _full((D, OD2)),
              _full((1, OD2)),
              _full((1, OD2)),
              _full((1, OD2)),
              _full((OD2, OD2)),
              _full((OD2, OD2)),
              pl.BlockSpec((32, OD2), lambda i: (i % 8, 0))],
    out_specs=[pl.BlockSpec((32, OD2), lambda i: (i, 0)),
               pl.BlockSpec((32, OD2), lambda i: (i, 0))],
    out_shape=[jax.ShapeDtypeStruct((512, OD2), jnp.float32),
               jax.ShapeDtypeStruct((512, OD2), jnp.float32)],
)

_final = pl.pallas_call(
    _final_body,
    out_shape=jax.ShapeDtypeStruct((256, 2), jnp.float32),
)


def kernel(x1, x2, cell, edge_index1, edge_index2, batch1, batch2,
           Wc1, bc1, Wc2, bc2, Wc3, bc3, g2, bt2, g4, bt4,
           Wr1, br1, Wr2, br2, Wr3, br3, Wq, Wk, Wv, Wg1, bg1, Wf1, bf1,
           Wo, bo):
    srcs = jnp.concatenate([edge_index1[0], edge_index2[0] + N]).reshape(-1, W)
    dsts = jnp.concatenate([edge_index1[1], edge_index2[1]]).reshape(-1, W)
    x12 = jnp.concatenate([x1, x2], axis=0)
    ones_h = jnp.ones((W,), jnp.float32)
    zrow_h = jnp.zeros((RPT,), jnp.float32)

    _deg, _spmm = _sc_kernels()
    deg = _deg(dsts, ones_h, zrow_h).reshape(NTOW * N, 1)
    cv, q = _cell(cell, Wr1, br1.reshape(1, -1), Wr2, br2.reshape(1, -1),
                  Wr3, br3.reshape(1, -1), Wq)
    u1, dinv = _pre(x12, Wc1, deg)
    y1 = _spmm(u1, srcs, dsts)
    u2 = _mid1(y1, dinv, bc1.reshape(1, -1), Wc2)
    y2 = _spmm(u2, srcs, dsts)
    u3 = _mid2(y2, dinv, bc2.reshape(1, -1))
    y3 = _spmm(u3, srcs, dsts)
    Wg1p = Wg1.reshape(64, 4, OD2).transpose(1, 0, 2).reshape(OD2, OD2)
    ctx, pool = _head(y3, dinv, Wc3, bc3.reshape(1, -1),
                      g2.reshape(1, -1), bt2.reshape(1, -1), Wk, Wv, q)
    return _final(ctx, pool, cv, Wg1p, Wg1, bg1.reshape(1, -1),
                  g4.reshape(1, -1), bt4.reshape(1, -1), Wf1,
                  bf1.reshape(1, -1), Wo, bo.reshape(1, -1))


# speedup vs baseline: 1.0199x; 1.0199x over previous
import functools

import jax
import jax.numpy as jnp
from jax import lax
from jax.experimental import pallas as pl
from jax.experimental.pallas import tpu as pltpu
from jax.experimental.pallas import tpu_sc as plsc

N = 10240
E = 163840
D = 128
OD2 = 256
NTOW = 2
NC, NS = 2, 16
W = 128
EPT = E // NS
NWIN = EPT // W
RPT = N // NS
DEPTH = 2
CH = 16
NCHUNK = NWIN // CH
BNS = 0.9999950000374996


def _deg_body(dsts2, ones_h, zrow_h, deg, dstt, onesv, zv, acc):
    c = lax.axis_index("c")
    s = lax.axis_index("s")
    pltpu.sync_copy(ones_h, onesv)
    pltpu.sync_copy(zrow_h, zv)
    pltpu.sync_copy(zv, acc.at[pl.ds(s * RPT, RPT)])
    tb = (c * NS + s) * NWIN
    pltpu.sync_copy(dsts2.at[pl.ds(tb, NWIN)], dstt)
    plsc.subcore_barrier()

    def body(w, carry):
        pltpu.sync_copy(onesv, acc.at[dstt.at[w]], add=True)
        return carry

    lax.fori_loop(0, NWIN, body, 0)
    plsc.subcore_barrier()
    pltpu.sync_copy(acc.at[pl.ds(s * RPT, RPT)],
                    deg.at[pl.ds(c * N + s * RPT, RPT)])


def _spmm_body(xs, srcs2, dsts2, y, srct, dstt, rows, acc, sems):
    c = lax.axis_index("c")
    s = lax.axis_index("s")
    rbase = s * RPT
    pltpu.sync_copy(xs.at[pl.ds(c * N + rbase, RPT)],
                    acc.at[pl.ds(rbase, RPT)])
    plsc.subcore_barrier()
    tb = (c * NS + s) * NWIN

    def chunk_body(ch, carry):
        pltpu.sync_copy(srcs2.at[pl.ds(tb + ch * CH, CH)], srct)
        pltpu.sync_copy(dsts2.at[pl.ds(tb + ch * CH, CH)], dstt)
        for k in range(DEPTH - 1):
            pltpu.async_copy(xs.at[srct.at[k]], rows[k], sems[k])
        for w in range(CH):
            b = w % DEPTH
            nb = (b + DEPTH - 1) % DEPTH
            if w + DEPTH - 1 < CH:
                pltpu.async_copy(xs.at[srct.at[w + DEPTH - 1]],
                                 rows[nb], sems[nb])
            pltpu.make_async_copy(xs.at[srct.at[w]], rows[b], sems[b]).wait()
            pltpu.sync_copy(rows[b], acc.at[dstt.at[w]], add=True)
        return carry

    lax.fori_loop(0, NCHUNK, chunk_body, 0)
    plsc.subcore_barrier()
    pltpu.sync_copy(acc.at[pl.ds(rbase, RPT)], y.at[pl.ds(c * N + rbase, RPT)])


@functools.cache
def _sc_kernels():
    mesh = plsc.VectorSubcoreMesh(
        core_axis_name="c", subcore_axis_name="s",
        num_cores=NC, num_subcores=NS)
    deg = pl.kernel(
        _deg_body,
        out_type=jax.ShapeDtypeStruct((NTOW * N,), jnp.float32),
        mesh=mesh,
        scratch_types=[
            pltpu.VMEM((NWIN, W), jnp.int32),
            pltpu.VMEM((W,), jnp.float32),
            pltpu.VMEM((RPT,), jnp.float32),
            pltpu.VMEM_SHARED((N,), jnp.float32),
        ],
    )
    spmm = pl.kernel(
        _spmm_body,
        out_type=jax.ShapeDtypeStruct((NTOW * N, D), jnp.float32),
        mesh=mesh,
        scratch_types=[
            pltpu.VMEM((CH, W), jnp.int32),
            pltpu.VMEM((CH, W), jnp.int32),
            [pltpu.VMEM((W, D), jnp.float32) for _ in range(DEPTH)],
            pltpu.VMEM_SHARED((N, D), jnp.float32),
            [pltpu.SemaphoreType.DMA for _ in range(DEPTH)],
        ],
    )
    return deg, spmm


def _cell_body(cell, Wr1, br1, Wr2, br2, Wr3, br3, Wq, cv_out, q_out):
    x = cell[...]
    nrm = jnp.sqrt(jnp.sum(x * x, axis=1, keepdims=True))
    x = x / jnp.maximum(nrm, 1e-12)
    x = jnp.maximum(
        jnp.dot(x, Wr1[...], preferred_element_type=jnp.float32) + br1[...], 0.0)
    x = jnp.maximum(
        jnp.dot(x, Wr2[...], preferred_element_type=jnp.float32) + br2[...], 0.0)
    cv = jnp.dot(x, Wr3[...], preferred_element_type=jnp.float32) + br3[...]
    cv_out[...] = cv
    q_out[...] = jnp.dot(cv, Wq[...], preferred_element_type=jnp.float32)


def _pre_body(x, Wc1, deg, u1, dinv_out):
    dinv = 1.0 / jnp.sqrt(deg[...] + 1.0)
    t = jnp.dot(x[...], Wc1[...], preferred_element_type=jnp.float32)
    u1[...] = t * dinv
    dinv_out[...] = dinv


def _mid1_body(y, dinv, bc1, Wc2, u2):
    h = jnp.maximum(y[...] * dinv[...] + bc1[...], 0.0)
    u2[...] = jnp.dot(h, Wc2[...], preferred_element_type=jnp.float32) * dinv[...]


def _mid2_body(y, dinv, bc2, u3):
    h = jnp.maximum(y[...] * dinv[...] + bc2[...], 0.0)
    u3[...] = h * dinv[...]


def _head_body(y, dinv, Wc3, bc3, g2, bt2, Wk, Wv, q, ctx_out, pool_out):
    agg = y[...] * dinv[...]
    h3 = jnp.dot(agg, Wc3[...], preferred_element_type=jnp.float32) + bc3[...]
    h3 = jnp.maximum(g2[...] * h3 * BNS + bt2[...], 0.0)
    pool_out[...] = jnp.max(h3.reshape(32, 40, OD2), axis=1)
    K = jnp.dot(h3, Wk[...], preferred_element_type=jnp.float32)
    V = jnp.dot(h3, Wv[...], preferred_element_type=jnp.float32)
    qb = q[...]
    ctxs = []
    for h in range(4):
        Kh = K[:, h * 64:(h + 1) * 64].reshape(32, 40, 64)
        Vh = V[:, h * 64:(h + 1) * 64].reshape(32, 40, 64)
        qh = qb[:, h * 64:(h + 1) * 64]
        s = jnp.sum(Kh * qh[:, None, :], axis=-1) * 0.0625
        m = jnp.max(s, axis=-1, keepdims=True)
        e = jnp.exp(s - m)
        p = e / jnp.sum(e, axis=-1, keepdims=True)
        ctxs.append(jnp.sum(p[:, :, None] * Vh, axis=1))
    ctx_out[...] = jnp.concatenate(ctxs, axis=1)


def _final_body(ctx, pool, cv, Wg1p, Wg1, bg1, g4, bt4, Wf1, bf1, Wo, bo, out):
    ctxa = ctx[...]
    poola = pool[...]
    ctxs = ctxa[0:256] + ctxa[256:512]
    pools = poola[0:256] + poola[256:512]
    drug = (jnp.dot(ctxs, Wg1p[...], preferred_element_type=jnp.float32)
            + jnp.dot(pools, Wg1[...], preferred_element_type=jnp.float32)
            + 2.0 * bg1[...])
    xc = jnp.concatenate([drug, cv[...]], axis=1)
    nrm = jnp.sqrt(jnp.sum(xc * xc, axis=1, keepdims=True))
    xc = xc / jnp.maximum(nrm, 1e-12)
    xc = jnp.maximum(g4[...] * xc * BNS + bt4[...], 0.0)
    xc = jnp.maximum(
        jnp.dot(xc, Wf1[...], preferred_element_type=jnp.float32) + bf1[...], 0.0)
    out[...] = jnp.dot(xc, Wo[...], preferred_element_type=jnp.float32) + bo[...]


def _full(shape):
    return pl.BlockSpec(shape, lambda *_: tuple(0 for _ in shape))


_cell = pl.pallas_call(
    _cell_body,
    out_shape=[jax.ShapeDtypeStruct((256, OD2), jnp.float32),
               jax.ShapeDtypeStruct((256, OD2), jnp.float32)],
)

_pre = pl.pallas_call(
    _pre_body,
    grid=(NTOW * N // 256,),
    in_specs=[pl.BlockSpec((256, 78), lambda i: (i, 0)),
              _full((78, D)),
              pl.BlockSpec((256, 1), lambda i: (i, 0))],
    out_specs=[pl.BlockSpec((256, D), lambda i: (i, 0)),
               pl.BlockSpec((256, 1), lambda i: (i, 0))],
    out_shape=[jax.ShapeDtypeStruct((NTOW * N, D), jnp.float32),
               jax.ShapeDtypeStruct((NTOW * N, 1), jnp.float32)],
)

_mid1 = pl.pallas_call(
    _mid1_body,
    grid=(NTOW * N // 256,),
    in_specs=[pl.BlockSpec((256, D), lambda i: (i, 0)),
              pl.BlockSpec((256, 1), lambda i: (i, 0)),
              _full((1, D)),
              _full((D, D))],
    out_specs=pl.BlockSpec((256, D), lambda i: (i, 0)),
    out_shape=jax.ShapeDtypeStruct((NTOW * N, D), jnp.float32),
)

_mid2 = pl.pallas_call(
    _mid2_body,
    grid=(NTOW * N // 256,),
    in_specs=[pl.BlockSpec((256, D), lambda i: (i, 0)),
              pl.BlockSpec((256, 1), lambda i: (i, 0)),
              _full((1, D))],
    out_specs=pl.BlockSpec((256, D), lambda i: (i, 0)),
    out_shape=jax.ShapeDtypeStruct((NTOW * N, D), jnp.float32),
)

_head = pl.pallas_call(
    _head_body,
    grid=(16,),
    in_specs=[pl.BlockSpec((1280, D), lambda i: (i, 0)),
              pl.BlockSpec((1280, 1), lambda i: (i, 0)),
              _full((D, OD2)),
              _full((1, OD2)),
              _full((1, OD2)),
              _full((1, OD2)),
              _full((OD2, OD2)),
              _full((OD2, OD2)),
              pl.BlockSpec((32, OD2), lambda i: (i % 8, 0))],
    out_specs=[pl.BlockSpec((32, OD2), lambda i: (i, 0)),
               pl.BlockSpec((32, OD2), lambda i: (i, 0))],
    out_shape=[jax.ShapeDtypeStruct((512, OD2), jnp.float32),
               jax.ShapeDtypeStruct((512, OD2), jnp.float32)],
)

_final = pl.pallas_call(
    _final_body,
    out_shape=jax.ShapeDtypeStruct((256, 2), jnp.float32),
)


def kernel(x1, x2, cell, edge_index1, edge_index2, batch1, batch2,
           Wc1, bc1, Wc2, bc2, Wc3, bc3, g2, bt2, g4, bt4,
           Wr1, br1, Wr2, br2, Wr3, br3, Wq, Wk, Wv, Wg1, bg1, Wf1, bf1,
           Wo, bo):
    srcs = jnp.concatenate([edge_index1[0], edge_index2[0] + N]).reshape(-1, W)
    dsts = jnp.concatenate([edge_index1[1], edge_index2[1]]).reshape(-1, W)
    x12 = jnp.concatenate([x1, x2], axis=0)
    ones_h = jnp.ones((W,), jnp.float32)
    zrow_h = jnp.zeros((RPT,), jnp.float32)

    _deg, _spmm = _sc_kernels()
    deg = _deg(dsts, ones_h, zrow_h).reshape(NTOW * N, 1)
    cv, q = _cell(cell, Wr1, br1.reshape(1, -1), Wr2, br2.reshape(1, -1),
                  Wr3, br3.reshape(1, -1), Wq)
    u1, dinv = _pre(x12, Wc1, deg)
    y1 = _spmm(u1, srcs, dsts)
    u2 = _mid1(y1, dinv, bc1.reshape(1, -1), Wc2)
    y2 = _spmm(u2, srcs, dsts)
    u3 = _mid2(y2, dinv, bc2.reshape(1, -1))
    y3 = _spmm(u3, srcs, dsts)
    Wg1p = Wg1.reshape(64, 4, OD2).transpose(1, 0, 2).reshape(OD2, OD2)
    ctx, pool = _head(y3, dinv, Wc3, bc3.reshape(1, -1),
                      g2.reshape(1, -1), bt2.reshape(1, -1), Wk, Wv, q)
    return _final(ctx, pool, cv, Wg1p, Wg1, bg1.reshape(1, -1),
                  g4.reshape(1, -1), bt4.reshape(1, -1), Wf1,
                  bf1.reshape(1, -1), Wo, bo.reshape(1, -1))
